# Initial kernel scaffold; baseline (speedup 1.0000x reference)
import jax
import jax.numpy as jnp
from jax import lax
from jax.experimental import pallas as pl
from jax.experimental.pallas import tpu as pltpu
from jax.experimental.pallas import tpu_sc as plsc

_N = 10000
_E = 320000
_D = 128
_NC = 2
_NS = 16
_EPW = _E // (_NC * _NS)
_K = 80
_NCHUNK = _EPW // _K
_RPT = _N // _NS


def _seg_body(table, gidx, sidx, zfeat, zcnt, part, cnt,
              gidx_v, sidx_v, rows_v, ones_v, acc, cntacc, sem):
  c = lax.axis_index("c")
  s = lax.axis_index("s")
  wid = c * _NS + s

  for i in range(_K // 16):
    ones_v[pl.ds(i * 16, 16)] = jnp.ones((16,), jnp.float32)

  pltpu.sync_copy(zfeat.at[pl.ds(s * _RPT, _RPT)], acc.at[pl.ds(s * _RPT, _RPT)])

  @pl.when(s == 0)
  def _():
    pltpu.sync_copy(zcnt, cntacc)

  plsc.subcore_barrier()

  def step(j, carry):
    base = wid * _EPW + j * _K
    pltpu.sync_copy(gidx.at[pl.ds(base, _K)], gidx_v)
    pltpu.sync_copy(sidx.at[pl.ds(base, _K)], sidx_v)
    pltpu.async_copy(table.at[gidx_v], rows_v, sem).wait()
    pltpu.sync_copy(rows_v, acc.at[sidx_v], add=True)
    pltpu.sync_copy(ones_v, cntacc.at[sidx_v], add=True)
    return carry

  lax.fori_loop(0, _NCHUNK, step, 0)

  plsc.subcore_barrier()
  pltpu.sync_copy(acc.at[pl.ds(s * _RPT, _RPT)],
                  part.at[c, pl.ds(s * _RPT, _RPT)])

  @pl.when(s == 0)
  def _():
    pltpu.sync_copy(cntacc, cnt.at[c])


_seg_sum = pl.kernel(
    _seg_body,
    out_type=(jax.ShapeDtypeStruct((_NC, _N, _D), jnp.float32),
              jax.ShapeDtypeStruct((_NC, _N), jnp.float32)),
    mesh=plsc.VectorSubcoreMesh(core_axis_name="c", subcore_axis_name="s"),
    scratch_types=[
        pltpu.VMEM((_K,), jnp.int32),
        pltpu.VMEM((_K,), jnp.int32),
        pltpu.VMEM((_K, _D), jnp.float32),
        pltpu.VMEM((_K,), jnp.float32),
        pltpu.VMEM_SHARED((_N, _D), jnp.float32),
        pltpu.VMEM_SHARED((_N,), jnp.float32),
        pltpu.SemaphoreType.DMA,
    ],
)


def _mm_body(x_ref, w_ref, b_ref, o_ref):
  o_ref[...] = lax.dot_general(
      x_ref[...], w_ref[...], (((1,), (1,)), ((), ())),
      preferred_element_type=jnp.float32) + b_ref[...]


_mm = pl.pallas_call(
    _mm_body,
    out_shape=jax.ShapeDtypeStruct((_N, _D), jnp.float32),
)


def _comb_body(p_ref, c_ref, o_ref):
  ssum = p_ref[0] + p_ref[1]
  csum = jnp.maximum(c_ref[0] + c_ref[1], 1.0)
  o_ref[...] = ssum / csum[:, None]


_combine = pl.pallas_call(
    _comb_body,
    out_shape=jax.ShapeDtypeStruct((_N, _D), jnp.float32),
)


def kernel(x, hyperedge_index, W, b):
  nodes = hyperedge_index[0]
  edges = hyperedge_index[1]
  h = _mm(x, W, b.reshape(1, _D))
  zf = jnp.zeros((_N, _D), jnp.float32)
  zc = jnp.zeros((_N,), jnp.float32)
  part_e, cnt_e = _seg_sum(h, nodes, edges, zf, zc)
  ef = _combine(part_e, cnt_e)
  part_n, cnt_n = _seg_sum(ef, edges, nodes, zf, zc)
  return _combine(part_n, cnt_n)


# speedup vs baseline: 4.9194x; 4.9194x over previous
import jax
import jax.numpy as jnp
from jax import lax
from jax.experimental import pallas as pl
from jax.experimental.pallas import tpu as pltpu
from jax.experimental.pallas import tpu_sc as plsc

_N = 10000
_E = 320000
_D = 128
_NC = 2
_NS = 16
_EPW = _E // (_NC * _NS)
_K = 80
_NCHUNK = _EPW // _K
_ST = 624
_RB = _NS * _ST
_REM = _N - _RB


def _seg_body(table, gidx, sidx, zfeat, zcnt, part, cnt,
              gidx_v, sidx_v, rows_v, ones_v, acc, cntacc, sem):
  c = lax.axis_index("c")
  s = lax.axis_index("s")
  wid = c * _NS + s

  for i in range(_K // 16):
    ones_v[pl.ds(i * 16, 16)] = jnp.ones((16,), jnp.float32)

  pltpu.sync_copy(zfeat.at[pl.ds(s * _ST, _ST)], acc.at[pl.ds(s * _ST, _ST)])

  @pl.when(s == 0)
  def _():
    pltpu.sync_copy(zfeat.at[pl.ds(_RB, _REM)], acc.at[pl.ds(_RB, _REM)])
    pltpu.sync_copy(zcnt, cntacc)

  plsc.subcore_barrier()

  def step(j, carry):
    base = wid * _EPW + j * _K
    pltpu.sync_copy(gidx.at[pl.ds(base, _K)], gidx_v)
    pltpu.sync_copy(sidx.at[pl.ds(base, _K)], sidx_v)
    pltpu.async_copy(table.at[gidx_v], rows_v, sem).wait()
    pltpu.sync_copy(rows_v, acc.at[sidx_v], add=True)
    pltpu.sync_copy(ones_v, cntacc.at[sidx_v], add=True)
    return carry

  lax.fori_loop(0, _NCHUNK, step, 0)

  plsc.subcore_barrier()
  pltpu.sync_copy(acc.at[pl.ds(s * _ST, _ST)],
                  part.at[c, pl.ds(s * _ST, _ST)])

  @pl.when(s == 0)
  def _():
    pltpu.sync_copy(acc.at[pl.ds(_RB, _REM)], part.at[c, pl.ds(_RB, _REM)])
    pltpu.sync_copy(cntacc, cnt.at[c, 0])


_seg_sum = pl.kernel(
    _seg_body,
    out_type=(jax.ShapeDtypeStruct((_NC, _N, _D), jnp.float32),
              jax.ShapeDtypeStruct((_NC, 1, _N), jnp.float32)),
    mesh=plsc.VectorSubcoreMesh(core_axis_name="c", subcore_axis_name="s"),
    scratch_types=[
        pltpu.VMEM((_K,), jnp.int32),
        pltpu.VMEM((_K,), jnp.int32),
        pltpu.VMEM((_K, _D), jnp.float32),
        pltpu.VMEM((_K,), jnp.float32),
        pltpu.VMEM_SHARED((_N, _D), jnp.float32),
        pltpu.VMEM_SHARED((_N,), jnp.float32),
        pltpu.SemaphoreType.DMA,
    ],
)


def _mm_body(x_ref, w_ref, b_ref, o_ref):
  o_ref[...] = lax.dot_general(
      x_ref[...], w_ref[...], (((1,), (1,)), ((), ())),
      preferred_element_type=jnp.float32) + b_ref[...]


_mm = pl.pallas_call(
    _mm_body,
    out_shape=jax.ShapeDtypeStruct((_N, _D), jnp.float32),
)


def _comb_body(p_ref, c_ref, o_ref):
  ssum = p_ref[0] + p_ref[1]
  csum = jnp.maximum(c_ref[0] + c_ref[1], 1.0)
  o_ref[...] = ssum / csum[:, None]


_combine = pl.pallas_call(
    _comb_body,
    out_shape=jax.ShapeDtypeStruct((_N, _D), jnp.float32),
)


def kernel(x, hyperedge_index, W, b):
  nodes = hyperedge_index[0]
  edges = hyperedge_index[1]
  h = _mm(x, W, b.reshape(1, _D))
  zf = jnp.zeros((_N, _D), jnp.float32)
  zc = jnp.zeros((_N,), jnp.float32)
  part_e, cnt_e = _seg_sum(h, nodes, edges, zf, zc)
  ef = _combine(part_e, cnt_e.reshape(_NC, _N))
  part_n, cnt_n = _seg_sum(ef, edges, nodes, zf, zc)
  return _combine(part_n, cnt_n.reshape(_NC, _N))
